# Initial kernel scaffold; baseline (speedup 1.0000x reference)
import jax
import jax.numpy as jnp
from jax.experimental import pallas as pl


def kernel(x_cat, delta_E, gene_E, W_seq, b_seq, W_func, b_func, geno_table, gamma, beta):
    raise NotImplementedError("write your pallas kernel here")


# speedup vs baseline: 3.7719x; 3.7719x over previous
import functools

import jax
import jax.numpy as jnp
from jax import lax
from jax.experimental import pallas as pl
from jax.experimental.pallas import tpu as pltpu
from jax.experimental.pallas import tpu_sc as plsc

N_SNPS = 20000
D_DELTA = 768
D_GENE = 200
D_MODEL = 64
BATCH = 32
N_GENO = 4
EPS = 1e-5


_SNP_BLK = 2000


def _p1_body(delta, gene, wseq, wfunc, bseq, bfunc, table, gamma, beta, out):
    bio = lax.dot_general(delta[...], wseq[...], (((1,), (1,)), ((), ())),
                          preferred_element_type=jnp.float32)
    bio = bio + lax.dot_general(gene[...], wfunc[...], (((1,), (1,)), ((), ())),
                                preferred_element_type=jnp.float32)
    bio = bio + bseq[...] + bfunc[...]
    g_row = gamma[...]
    b_row = beta[...]
    for g in range(N_GENO):
        t = bio + table[g, :][None, :]
        mu = jnp.mean(t, axis=-1, keepdims=True)
        var = jnp.mean(jnp.square(t - mu), axis=-1, keepdims=True)
        out[g] = (t - mu) * lax.rsqrt(var + EPS) * g_row + b_row


def _phase1(delta_E, gene_E, W_seq, b_seq, W_func, b_func, geno_table, gamma, beta):
    grid = (N_SNPS // _SNP_BLK,)
    full = lambda shape: pl.BlockSpec(shape, lambda i: tuple(0 for _ in shape))
    return pl.pallas_call(
        _p1_body,
        grid=grid,
        in_specs=[
            pl.BlockSpec((_SNP_BLK, D_DELTA), lambda i: (i, 0)),
            pl.BlockSpec((_SNP_BLK, D_GENE), lambda i: (i, 0)),
            full((D_MODEL, D_DELTA)),
            full((D_MODEL, D_GENE)),
            full((1, D_MODEL)),
            full((1, D_MODEL)),
            full((N_GENO, D_MODEL)),
            full((1, D_MODEL)),
            full((1, D_MODEL)),
        ],
        out_specs=pl.BlockSpec((N_GENO, _SNP_BLK, D_MODEL), lambda i: (0, i, 0)),
        out_shape=jax.ShapeDtypeStruct((N_GENO, N_SNPS, D_MODEL), jnp.float32),
    )(delta_E, gene_E, W_seq, W_func,
      b_seq.reshape(1, D_MODEL), b_func.reshape(1, D_MODEL), geno_table,
      gamma.reshape(1, D_MODEL), beta.reshape(1, D_MODEL))


_CHUNK = 800
_GSUB = 80
_NSUB = _CHUNK // _GSUB


def _make_gather():
    info = plsc.get_sparse_core_info()
    nc, ns = info.num_cores, info.num_subcores
    nw = nc * ns
    rows_per_w = (BATCH * N_SNPS) // nw
    n_chunks = rows_per_w // _CHUNK

    mesh = plsc.VectorSubcoreMesh(core_axis_name="c", subcore_axis_name="s")

    @functools.partial(
        pl.kernel,
        mesh=mesh,
        out_type=jax.ShapeDtypeStruct((BATCH * N_SNPS, D_MODEL), jnp.float32),
        scratch_types=[
            pltpu.VMEM((_CHUNK,), jnp.int32),
            pltpu.VMEM((_NSUB, _GSUB), jnp.int32),
            pltpu.VMEM((_CHUNK, D_MODEL), jnp.float32),
            pltpu.SemaphoreType.DMA,
        ],
        compiler_params=pltpu.CompilerParams(use_tc_tiling_on_sc=False),
    )
    def gather_kernel(a_hbm, x_hbm, out_hbm, x_v, idx_v, rows_v, sem):
        wid = lax.axis_index("s") * nc + lax.axis_index("c")
        base = wid * rows_per_w

        def chunk_body(k, carry):
            off = base + k * _CHUNK
            snp0 = k * _CHUNK
            pltpu.sync_copy(x_hbm.at[pl.ds(off, _CHUNK)], x_v)
            lane = lax.iota(jnp.int32, 16)
            for j in range(_NSUB):
                for c in range(_GSUB // 16):
                    x16 = x_v[pl.ds(j * _GSUB + c * 16, 16)]
                    snp = lane + (snp0 + j * _GSUB + c * 16)
                    idx_v[j, pl.ds(c * 16, 16)] = x16 * N_SNPS + snp
            copies = [
                pltpu.async_copy(a_hbm.at[idx_v.at[j]],
                                 rows_v.at[pl.ds(j * _GSUB, _GSUB)], sem)
                for j in range(_NSUB)
            ]
            for cp in copies:
                cp.wait()
            pltpu.sync_copy(rows_v, out_hbm.at[pl.ds(off, _CHUNK)])
            return carry

        lax.fori_loop(0, n_chunks, chunk_body, 0)

    return gather_kernel


def kernel(x_cat, delta_E, gene_E, W_seq, b_seq, W_func, b_func, geno_table, gamma, beta):
    a = _phase1(delta_E, gene_E, W_seq, b_seq, W_func, b_func,
                geno_table, gamma, beta)
    a_flat = a.reshape(N_GENO * N_SNPS, D_MODEL)
    x_flat = x_cat.reshape(BATCH * N_SNPS)
    out_flat = _make_gather()(a_flat, x_flat)
    return out_flat.reshape(BATCH, N_SNPS, D_MODEL)


# speedup vs baseline: 21.7801x; 5.7743x over previous
import jax
import jax.numpy as jnp
from jax import lax
from jax.experimental import pallas as pl

N_SNPS = 20000
D_DELTA = 768
D_GENE = 200
D_MODEL = 64
BATCH = 32
N_GENO = 4
EPS = 1e-5

_S = 512


def _body(x_blk, delta, gene, wseq, wfunc, bseq, bfunc, table, gamma, beta, out):
    bio = lax.dot_general(wseq[...], delta[...], (((1,), (1,)), ((), ())),
                          preferred_element_type=jnp.float32)
    bio = bio + lax.dot_general(wfunc[...], gene[...], (((1,), (1,)), ((), ())),
                                preferred_element_type=jnp.float32)
    bio = bio + (bseq[...] + bfunc[...]).reshape(D_MODEL, 1)
    g_col = gamma[...].reshape(D_MODEL, 1)
    b_col = beta[...].reshape(D_MODEL, 1)
    a = []
    for g in range(N_GENO):
        t = bio + table[...][g, :].reshape(D_MODEL, 1)
        mu = jnp.mean(t, axis=0, keepdims=True)
        var = jnp.mean(jnp.square(t - mu), axis=0, keepdims=True)
        a.append((t - mu) * lax.rsqrt(var + EPS) * g_col + b_col)
    x = x_blk[...]
    for b in range(BATCH):
        xb = x[b, :].reshape(1, _S)
        sel = jnp.where(xb == 2, a[2], a[3])
        sel = jnp.where(xb == 1, a[1], sel)
        sel = jnp.where(xb == 0, a[0], sel)
        out[b] = sel


def kernel(x_cat, delta_E, gene_E, W_seq, b_seq, W_func, b_func, geno_table, gamma, beta):
    grid = (pl.cdiv(N_SNPS, _S),)
    full = lambda shape: pl.BlockSpec(shape, lambda i: tuple(0 for _ in shape))
    out_t = pl.pallas_call(
        _body,
        grid=grid,
        in_specs=[
            pl.BlockSpec((BATCH, _S), lambda i: (0, i)),
            pl.BlockSpec((_S, D_DELTA), lambda i: (i, 0)),
            pl.BlockSpec((_S, D_GENE), lambda i: (i, 0)),
            full((D_MODEL, D_DELTA)),
            full((D_MODEL, D_GENE)),
            full((1, D_MODEL)),
            full((1, D_MODEL)),
            full((N_GENO, D_MODEL)),
            full((1, D_MODEL)),
            full((1, D_MODEL)),
        ],
        out_specs=pl.BlockSpec((BATCH, D_MODEL, _S), lambda i: (0, 0, i)),
        out_shape=jax.ShapeDtypeStruct((BATCH, D_MODEL, N_SNPS), jnp.float32),
    )(x_cat, delta_E, gene_E, W_seq, W_func,
      b_seq.reshape(1, D_MODEL), b_func.reshape(1, D_MODEL), geno_table,
      gamma.reshape(1, D_MODEL), beta.reshape(1, D_MODEL))
    return jnp.transpose(out_t, (0, 2, 1))


# speedup vs baseline: 24.6958x; 1.1339x over previous
import jax
import jax.numpy as jnp
from jax import lax
from jax.experimental import pallas as pl

N_SNPS = 20000
D_DELTA = 768
D_GENE = 200
D_MODEL = 64
BATCH = 32
N_GENO = 4
EPS = 1e-5

_S = 1024


def _body(x_blk, delta, gene, wseq, wfunc, bseq, bfunc, table, gamma, beta, out):
    bio = lax.dot_general(wseq[...], delta[...], (((1,), (1,)), ((), ())),
                          preferred_element_type=jnp.float32)
    bio = bio + lax.dot_general(wfunc[...], gene[...], (((1,), (1,)), ((), ())),
                                preferred_element_type=jnp.float32)
    bio = bio + (bseq[...] + bfunc[...]).reshape(D_MODEL, 1)
    g_col = gamma[...].reshape(D_MODEL, 1)
    b_col = beta[...].reshape(D_MODEL, 1)
    a = []
    for g in range(N_GENO):
        t = bio + table[...][g, :].reshape(D_MODEL, 1)
        mu = jnp.mean(t, axis=0, keepdims=True)
        var = jnp.mean(jnp.square(t - mu), axis=0, keepdims=True)
        a.append((t - mu) * lax.rsqrt(var + EPS) * g_col + b_col)
    x = x_blk[...]
    for b in range(BATCH):
        xb = x[b, :].reshape(1, _S)
        sel = jnp.where(xb == 2, a[2], a[3])
        sel = jnp.where(xb == 1, a[1], sel)
        sel = jnp.where(xb == 0, a[0], sel)
        out[b] = sel


def kernel(x_cat, delta_E, gene_E, W_seq, b_seq, W_func, b_func, geno_table, gamma, beta):
    grid = (pl.cdiv(N_SNPS, _S),)
    full = lambda shape: pl.BlockSpec(shape, lambda i: tuple(0 for _ in shape))
    out_t = pl.pallas_call(
        _body,
        grid=grid,
        in_specs=[
            pl.BlockSpec((BATCH, _S), lambda i: (0, i)),
            pl.BlockSpec((_S, D_DELTA), lambda i: (i, 0)),
            pl.BlockSpec((_S, D_GENE), lambda i: (i, 0)),
            full((D_MODEL, D_DELTA)),
            full((D_MODEL, D_GENE)),
            full((1, D_MODEL)),
            full((1, D_MODEL)),
            full((N_GENO, D_MODEL)),
            full((1, D_MODEL)),
            full((1, D_MODEL)),
        ],
        out_specs=pl.BlockSpec((BATCH, D_MODEL, _S), lambda i: (0, 0, i)),
        out_shape=jax.ShapeDtypeStruct((BATCH, D_MODEL, N_SNPS), jnp.float32),
    )(x_cat, delta_E, gene_E, W_seq, W_func,
      b_seq.reshape(1, D_MODEL), b_func.reshape(1, D_MODEL), geno_table,
      gamma.reshape(1, D_MODEL), beta.reshape(1, D_MODEL))
    return jnp.transpose(out_t, (0, 2, 1))


# speedup vs baseline: 25.1897x; 1.0200x over previous
import jax
import jax.numpy as jnp
from jax import lax
from jax.experimental import pallas as pl

N_SNPS = 20000
D_DELTA = 768
D_GENE = 200
D_MODEL = 64
BATCH = 32
N_GENO = 4
EPS = 1e-5

_S = 2048


def _body(x_blk, delta, gene, wseq, wfunc, bseq, bfunc, table, gamma, beta, out):
    bio = lax.dot_general(wseq[...], delta[...], (((1,), (1,)), ((), ())),
                          preferred_element_type=jnp.float32)
    bio = bio + lax.dot_general(wfunc[...], gene[...], (((1,), (1,)), ((), ())),
                                preferred_element_type=jnp.float32)
    bio = bio + (bseq[...] + bfunc[...]).reshape(D_MODEL, 1)
    g_col = gamma[...].reshape(D_MODEL, 1)
    b_col = beta[...].reshape(D_MODEL, 1)
    a = []
    for g in range(N_GENO):
        t = bio + table[...][g, :].reshape(D_MODEL, 1)
        mu = jnp.mean(t, axis=0, keepdims=True)
        var = jnp.mean(jnp.square(t - mu), axis=0, keepdims=True)
        a.append((t - mu) * lax.rsqrt(var + EPS) * g_col + b_col)
    x = x_blk[...]
    for b in range(BATCH):
        xb = x[b, :].reshape(1, _S)
        sel = jnp.where(xb == 2, a[2], a[3])
        sel = jnp.where(xb == 1, a[1], sel)
        sel = jnp.where(xb == 0, a[0], sel)
        out[b] = sel


def kernel(x_cat, delta_E, gene_E, W_seq, b_seq, W_func, b_func, geno_table, gamma, beta):
    grid = (pl.cdiv(N_SNPS, _S),)
    full = lambda shape: pl.BlockSpec(shape, lambda i: tuple(0 for _ in shape))
    out_t = pl.pallas_call(
        _body,
        grid=grid,
        in_specs=[
            pl.BlockSpec((BATCH, _S), lambda i: (0, i)),
            pl.BlockSpec((_S, D_DELTA), lambda i: (i, 0)),
            pl.BlockSpec((_S, D_GENE), lambda i: (i, 0)),
            full((D_MODEL, D_DELTA)),
            full((D_MODEL, D_GENE)),
            full((1, D_MODEL)),
            full((1, D_MODEL)),
            full((N_GENO, D_MODEL)),
            full((1, D_MODEL)),
            full((1, D_MODEL)),
        ],
        out_specs=pl.BlockSpec((BATCH, D_MODEL, _S), lambda i: (0, 0, i)),
        out_shape=jax.ShapeDtypeStruct((BATCH, D_MODEL, N_SNPS), jnp.float32),
    )(x_cat, delta_E, gene_E, W_seq, W_func,
      b_seq.reshape(1, D_MODEL), b_func.reshape(1, D_MODEL), geno_table,
      gamma.reshape(1, D_MODEL), beta.reshape(1, D_MODEL))
    return jnp.transpose(out_t, (0, 2, 1))
